# baseline (speedup 1.0000x reference)
import functools
import jax
import jax.numpy as jnp
from jax import lax
from jax.experimental import pallas as pl
from jax.experimental.pallas import tpu as pltpu
from jax.experimental.pallas import tpu_sc as plsc

B, N, D = 16, 8192, 3
KC = 128
KNN = 64
QB = 8
NROWS = B * KC
NSUB = 32
RPW = NROWS // NSUB
NBKT = 2048
CAP = 512
L = 16
NCH = N // L
BIGKEY = 0x7FFFFFFF


def _fps_body(xt_ref, c_ref):
    X = xt_ref[0]
    Y = xt_ref[1]
    Z = xt_ref[2]
    iota = jax.lax.broadcasted_iota(jnp.int32, (B, N), 1)

    px = X[:, 0:1]
    py = Y[:, 0:1]
    pz = Z[:, 0:1]
    c_ref[0:1] = jnp.concatenate([px, py, pz], axis=1).reshape(1, B, 3)

    dists0 = jnp.full((B, N), jnp.inf, dtype=jnp.float32)

    def step(k, carry):
        dists, px, py, pz = carry
        d = (X - px) ** 2 + (Y - py) ** 2 + (Z - pz) ** 2
        dists = jnp.minimum(dists, d)
        m = jnp.max(dists, axis=1, keepdims=True)
        idx = jnp.min(jnp.where(dists == m, iota, N), axis=1, keepdims=True)
        onehot = iota == idx
        px = jnp.sum(jnp.where(onehot, X, 0.0), axis=1, keepdims=True)
        py = jnp.sum(jnp.where(onehot, Y, 0.0), axis=1, keepdims=True)
        pz = jnp.sum(jnp.where(onehot, Z, 0.0), axis=1, keepdims=True)
        c_ref[pl.ds(k, 1)] = jnp.concatenate([px, py, pz], axis=1).reshape(1, B, 3)
        return dists, px, py, pz

    jax.lax.fori_loop(1, KC, step, (dists0, px, py, pz))


def _keys_body(xt_ref, c_ref, k_ref):
    X = xt_ref[0, 0:1, :]
    Y = xt_ref[0, 1:2, :]
    Z = xt_ref[0, 2:3, :]
    cx = c_ref[0, :, 0:1]
    cy = c_ref[0, :, 1:2]
    cz = c_ref[0, :, 2:3]

    def rb(v):
        return v.astype(jnp.bfloat16).astype(jnp.float32)
    cn = cx * cx + cy * cy + cz * cz
    xn = X * X + Y * Y + Z * Z
    dot = rb(cx) * rb(X) + rb(cy) * rb(Y) + rb(cz) * rb(Z)
    d2 = cn + xn - 2.0 * dot

    s = jax.lax.bitcast_convert_type(d2, jnp.int32)
    m = jnp.right_shift(s, 31)
    k_ref[...] = s ^ (m & BIGKEY)


def _sc_select(keys_hbm, xt_hbm, c_hbm, p_hbm,
               xb0, xb1, xb2, cq0, cq1, cq2,
               keyb0, keyb1, hist, pc, candk, candi, seli, outrow,
               sem0, sem1, semx):
    cid = lax.axis_index("c")
    sid = lax.axis_index("s")
    w = sid * 2 + cid
    b = w // 2
    q0 = (w % 2) * RPW
    r0 = b * KC + q0

    lanes = jax.lax.broadcasted_iota(jnp.int32, (L,), 0)
    ones16 = jnp.ones((L,), jnp.int32)

    pltpu.make_async_copy(xt_hbm.at[0, b], xb0, semx).start()
    pltpu.make_async_copy(xt_hbm.at[0, b], xb0, semx).wait()
    pltpu.make_async_copy(xt_hbm.at[1, b], xb1, semx).start()
    pltpu.make_async_copy(xt_hbm.at[1, b], xb1, semx).wait()
    pltpu.make_async_copy(xt_hbm.at[2, b], xb2, semx).start()
    pltpu.make_async_copy(xt_hbm.at[2, b], xb2, semx).wait()
    pltpu.make_async_copy(c_hbm.at[0, b, pl.ds(q0, RPW)], cq0, semx).start()
    pltpu.make_async_copy(c_hbm.at[0, b, pl.ds(q0, RPW)], cq0, semx).wait()
    pltpu.make_async_copy(c_hbm.at[1, b, pl.ds(q0, RPW)], cq1, semx).start()
    pltpu.make_async_copy(c_hbm.at[1, b, pl.ds(q0, RPW)], cq1, semx).wait()
    pltpu.make_async_copy(c_hbm.at[2, b, pl.ds(q0, RPW)], cq2, semx).start()
    pltpu.make_async_copy(c_hbm.at[2, b, pl.ds(q0, RPW)], cq2, semx).wait()

    pltpu.make_async_copy(keys_hbm.at[r0], keyb0, sem0).start()

    def process_row(jj, keyrow):
        def zstep(i, _):
            hist[pl.ds(i * L, L)] = jnp.zeros((L,), jnp.int32)
            return 0
        lax.fori_loop(0, NBKT // L, zstep, 0, unroll=8)

        def astep(i, mk):
            k = keyrow[pl.ds(i * L, L)]
            bkt = jnp.right_shift(k, 21) + 1024
            plsc.addupdate_scatter(hist, [bkt], ones16)
            return jnp.minimum(mk, k)
        mk = lax.fori_loop(0, NCH, astep, jnp.full((L,), BIGKEY), unroll=8)
        minkey = jnp.min(mk)
        gstart = jnp.right_shift(jnp.right_shift(minkey, 21) + 1024, 4)

        def wcond(c):
            return jnp.logical_not(c[2])

        def wbody(c):
            g, cum, _ = c
            sv = jnp.sum(hist[pl.ds(g * L, L)])
            hit = jnp.logical_or(cum + sv >= KNN, g >= NBKT // L - 1)
            return (jnp.where(hit, g, g + 1), jnp.where(hit, cum, cum + sv), hit)
        gstar, cumb, _ = lax.while_loop(
            wcond, wbody, (gstart, jnp.int32(0), False))
        vg = hist[pl.ds(gstar * L, L)]
        cin = plsc.cumsum(vg) + cumb
        crossed = cin >= KNN
        lane_b = jnp.min(jnp.where(crossed, lanes, jnp.int32(L)))
        bstar = gstar * L + jnp.minimum(lane_b, L - 1)

        def c1(i, _):
            k = keyrow[pl.ds(i * L, L)]
            bkt = jnp.right_shift(k, 21) + 1024
            pcv = plsc.all_reduce_population_count(bkt <= bstar)
            pc[pl.ds(i * L, L)] = pcv
            return 0
        lax.fori_loop(0, NCH, c1, 0, unroll=8)

        def c3(i, ptr):
            pcs = pc[pl.ds(i * L, L)][0]

            @pl.when(pcs > 0)
            def _():
                k = keyrow[pl.ds(i * L, L)]
                bkt = jnp.right_shift(k, 21) + 1024
                msk = bkt <= bstar
                pos = ptr + plsc.cumsum(msk.astype(jnp.int32)) - 1
                msk = jnp.logical_and(msk, pos < CAP)
                plsc.store_scatter(candk, [pos], k, mask=msk)
                plsc.store_scatter(candi, [pos], i * L + lanes, mask=msk)
            return ptr + pcs
        ptr = lax.fori_loop(0, NCH, c3, jnp.int32(0), unroll=4)
        ptr = jnp.minimum(ptr, jnp.int32(CAP))
        plsc.store_scatter(candk, [ptr + lanes], jnp.full((L,), BIGKEY))
        nv = jnp.right_shift(ptr + L - 1, 4)

        def e0(v, c):
            vm0, vm1 = c
            mv = jnp.min(candk[pl.ds(v * L, L)])
            vm0 = jnp.where(lanes == v, mv, vm0)
            vm1 = jnp.where(lanes == v - L, mv, vm1)
            return vm0, vm1
        vm = lax.fori_loop(0, nv, e0,
                           (jnp.full((L,), BIGKEY), jnp.full((L,), BIGKEY)))

        def estep(k, c):
            vm0, vm1 = c
            m = jnp.min(jnp.minimum(vm0, vm1))
            vstar = jnp.min(jnp.minimum(
                jnp.where(vm0 == m, lanes, jnp.int32(2 * L)),
                jnp.where(vm1 == m, lanes + L, jnp.int32(2 * L))))
            kv = candk[pl.ds(vstar * L, L)]
            iv = candi[pl.ds(vstar * L, L)]
            eq = kv == m
            im = jnp.min(jnp.where(eq, iv, jnp.int32(BIGKEY)))
            kv2 = jnp.where(jnp.logical_and(eq, iv == im), jnp.int32(BIGKEY), kv)
            candk[pl.ds(vstar * L, L)] = kv2
            nm = jnp.min(kv2)
            vm0 = jnp.where(lanes == vstar, nm, vm0)
            vm1 = jnp.where(lanes == vstar - L, nm, vm1)
            plsc.store_scatter(seli, [jnp.full((L,), k)],
                               jnp.full((L,), im), mask=lanes == 0)
            return vm0, vm1
        lax.fori_loop(0, KNN, estep, vm)

        cqx = plsc.load_gather(cq0, [jnp.full((L,), jj)])
        cqy = plsc.load_gather(cq1, [jnp.full((L,), jj)])
        cqz = plsc.load_gather(cq2, [jnp.full((L,), jj)])
        for g in range(KNN // L):
            idxv = seli[pl.ds(g * L, L)]
            pos = (g * L + lanes) * 3
            plsc.store_scatter(outrow, [pos],
                               plsc.load_gather(xb0, [idxv]) - cqx)
            plsc.store_scatter(outrow, [pos + 1],
                               plsc.load_gather(xb1, [idxv]) - cqy)
            plsc.store_scatter(outrow, [pos + 2],
                               plsc.load_gather(xb2, [idxv]) - cqz)
        pltpu.make_async_copy(outrow, p_hbm.at[r0 + jj], semx).start()
        pltpu.make_async_copy(outrow, p_hbm.at[r0 + jj], semx).wait()

    def rowpair(j2, _):
        jj0 = j2 * 2
        pltpu.make_async_copy(keys_hbm.at[r0 + jj0], keyb0, sem0).wait()

        @pl.when(jj0 + 1 < RPW)
        def _():
            pltpu.make_async_copy(keys_hbm.at[r0 + jj0 + 1], keyb1, sem1).start()
        process_row(jj0, keyb0)

        pltpu.make_async_copy(keys_hbm.at[r0 + jj0 + 1], keyb1, sem1).wait()

        @pl.when(jj0 + 2 < RPW)
        def _():
            pltpu.make_async_copy(keys_hbm.at[r0 + jj0 + 2], keyb0, sem0).start()
        process_row(jj0 + 1, keyb1)
        return 0

    lax.fori_loop(0, RPW // 2, rowpair, 0)


@jax.jit
def kernel(x):
    xt = jnp.transpose(x, (2, 0, 1))

    c_kb3 = pl.pallas_call(
        _fps_body,
        out_shape=jax.ShapeDtypeStruct((KC, B, 3), jnp.float32),
    )(xt)
    c = jnp.transpose(c_kb3, (1, 0, 2))

    nqb = KC // QB
    keys = pl.pallas_call(
        _keys_body,
        grid=(B, nqb),
        in_specs=[
            pl.BlockSpec((1, 3, N), lambda b, qb: (b, 0, 0)),
            pl.BlockSpec((1, QB, 3), lambda b, qb: (b, qb, 0)),
        ],
        out_specs=pl.BlockSpec((QB, N), lambda b, qb: (b * nqb + qb, 0)),
        out_shape=jax.ShapeDtypeStruct((NROWS, N), jnp.int32),
    )(jnp.transpose(x, (0, 2, 1)), c)

    c3bk = jnp.transpose(c_kb3, (2, 1, 0))

    mesh = plsc.VectorSubcoreMesh(core_axis_name="c", subcore_axis_name="s")
    sc = functools.partial(
        pl.kernel, mesh=mesh,
        out_type=jax.ShapeDtypeStruct((NROWS, KNN * 3), jnp.float32),
        scratch_types=[
            pltpu.VMEM((N,), jnp.float32),
            pltpu.VMEM((N,), jnp.float32),
            pltpu.VMEM((N,), jnp.float32),
            pltpu.VMEM((RPW,), jnp.float32),
            pltpu.VMEM((RPW,), jnp.float32),
            pltpu.VMEM((RPW,), jnp.float32),
            pltpu.VMEM((N,), jnp.int32),
            pltpu.VMEM((N,), jnp.int32),
            pltpu.VMEM((NBKT,), jnp.int32),
            pltpu.VMEM((N,), jnp.int32),
            pltpu.VMEM((CAP + L,), jnp.int32),
            pltpu.VMEM((CAP + L,), jnp.int32),
            pltpu.VMEM((KNN,), jnp.int32),
            pltpu.VMEM((KNN * 3,), jnp.float32),
            pltpu.SemaphoreType.DMA,
            pltpu.SemaphoreType.DMA,
            pltpu.SemaphoreType.DMA,
        ],
        compiler_params=pltpu.CompilerParams(needs_layout_passes=False),
    )(_sc_select)
    prows = sc(keys, xt, c3bk)

    p = prows.reshape(B, KC, KNN, 3)
    return (p, c)


# speedup vs baseline: 1.4273x; 1.4273x over previous
import functools
import jax
import jax.numpy as jnp
from jax import lax
from jax.experimental import pallas as pl
from jax.experimental.pallas import tpu as pltpu
from jax.experimental.pallas import tpu_sc as plsc

B, N, D = 16, 8192, 3
KC = 128
KNN = 64
QB = 8
NROWS = B * KC
NSUB = 32
RPW = NROWS // NSUB
NBKT = 2048
CAP = 512
L = 16
NCH = N // L
BIGKEY = 0x7FFFFFFF


def _fps_body(xt_ref, c_ref):
    X = xt_ref[0]
    Y = xt_ref[1]
    Z = xt_ref[2]
    iota = jax.lax.broadcasted_iota(jnp.int32, (B, N), 1)

    px = X[:, 0:1]
    py = Y[:, 0:1]
    pz = Z[:, 0:1]
    c_ref[0:1] = jnp.concatenate([px, py, pz], axis=1).reshape(1, B, 3)

    dists0 = jnp.full((B, N), jnp.inf, dtype=jnp.float32)

    def step(k, carry):
        dists, px, py, pz = carry
        d = (X - px) ** 2 + (Y - py) ** 2 + (Z - pz) ** 2
        dists = jnp.minimum(dists, d)
        m = jnp.max(dists, axis=1, keepdims=True)
        idx = jnp.min(jnp.where(dists == m, iota, N), axis=1, keepdims=True)
        onehot = iota == idx
        px = jnp.sum(jnp.where(onehot, X, 0.0), axis=1, keepdims=True)
        py = jnp.sum(jnp.where(onehot, Y, 0.0), axis=1, keepdims=True)
        pz = jnp.sum(jnp.where(onehot, Z, 0.0), axis=1, keepdims=True)
        c_ref[pl.ds(k, 1)] = jnp.concatenate([px, py, pz], axis=1).reshape(1, B, 3)
        return dists, px, py, pz

    jax.lax.fori_loop(1, KC, step, (dists0, px, py, pz))


def _keys_body(xt_ref, c_ref, k_ref):
    X = xt_ref[0, 0:1, :]
    Y = xt_ref[0, 1:2, :]
    Z = xt_ref[0, 2:3, :]
    cx = c_ref[0, :, 0:1]
    cy = c_ref[0, :, 1:2]
    cz = c_ref[0, :, 2:3]

    def rb(v):
        return v.astype(jnp.bfloat16).astype(jnp.float32)
    cn = cx * cx + cy * cy + cz * cz
    xn = X * X + Y * Y + Z * Z
    dot = rb(cx) * rb(X) + rb(cy) * rb(Y) + rb(cz) * rb(Z)
    d2 = cn + xn - 2.0 * dot

    s = jax.lax.bitcast_convert_type(d2, jnp.int32)
    m = jnp.right_shift(s, 31)
    k_ref[...] = s ^ (m & BIGKEY)


def _sc_select(keys_hbm, xt_hbm, c_hbm, p_hbm,
               xb0, xb1, xb2, cq0, cq1, cq2,
               keyb0, keyb1, hist, candk, candi, seli, outrow,
               sem0, sem1, semx):
    cid = lax.axis_index("c")
    sid = lax.axis_index("s")
    w = sid * 2 + cid
    b = w // 2
    q0 = (w % 2) * RPW
    r0 = b * KC + q0

    lanes = jax.lax.broadcasted_iota(jnp.int32, (L,), 0)
    ones16 = jnp.ones((L,), jnp.int32)

    pltpu.make_async_copy(xt_hbm.at[0, b], xb0, semx).start()
    pltpu.make_async_copy(xt_hbm.at[0, b], xb0, semx).wait()
    pltpu.make_async_copy(xt_hbm.at[1, b], xb1, semx).start()
    pltpu.make_async_copy(xt_hbm.at[1, b], xb1, semx).wait()
    pltpu.make_async_copy(xt_hbm.at[2, b], xb2, semx).start()
    pltpu.make_async_copy(xt_hbm.at[2, b], xb2, semx).wait()
    pltpu.make_async_copy(c_hbm.at[0, b, pl.ds(q0, RPW)], cq0, semx).start()
    pltpu.make_async_copy(c_hbm.at[0, b, pl.ds(q0, RPW)], cq0, semx).wait()
    pltpu.make_async_copy(c_hbm.at[1, b, pl.ds(q0, RPW)], cq1, semx).start()
    pltpu.make_async_copy(c_hbm.at[1, b, pl.ds(q0, RPW)], cq1, semx).wait()
    pltpu.make_async_copy(c_hbm.at[2, b, pl.ds(q0, RPW)], cq2, semx).start()
    pltpu.make_async_copy(c_hbm.at[2, b, pl.ds(q0, RPW)], cq2, semx).wait()

    pltpu.make_async_copy(keys_hbm.at[r0], keyb0, sem0).start()

    def process_row(jj, keyrow):
        def zstep(i, _):
            hist[pl.ds(i * L, L)] = jnp.zeros((L,), jnp.int32)
            return 0
        lax.fori_loop(0, NBKT // L, zstep, 0, unroll=8)

        def astep(i, mk):
            k = keyrow[pl.ds(i * L, L)]
            bkt = jnp.right_shift(k, 21) + 1024
            plsc.addupdate_scatter(hist, [bkt], ones16)
            return jnp.minimum(mk, k)
        mk = lax.fori_loop(0, NCH, astep, jnp.full((L,), BIGKEY), unroll=8)
        minkey = jnp.min(mk)
        gstart = jnp.right_shift(jnp.right_shift(minkey, 21) + 1024, 4)

        def wcond(c):
            return jnp.logical_not(c[2])

        def wbody(c):
            g, cum, _ = c
            sv = jnp.sum(hist[pl.ds(g * L, L)])
            hit = jnp.logical_or(cum + sv >= KNN, g >= NBKT // L - 1)
            return (jnp.where(hit, g, g + 1), jnp.where(hit, cum, cum + sv), hit)
        gstar, cumb, _ = lax.while_loop(
            wcond, wbody, (gstart, jnp.int32(0), False))
        vg = hist[pl.ds(gstar * L, L)]
        cin = plsc.cumsum(vg) + cumb
        crossed = cin >= KNN
        lane_b = jnp.min(jnp.where(crossed, lanes, jnp.int32(L)))
        bstar = gstar * L + jnp.minimum(lane_b, L - 1)

        def cstep(i, ptr_v):
            k = keyrow[pl.ds(i * L, L)]
            bkt = jnp.right_shift(k, 21) + 1024
            msk = bkt <= bstar
            pcv = plsc.all_reduce_population_count(msk)
            pos = ptr_v + plsc.cumsum(msk.astype(jnp.int32)) - 1
            msk = jnp.logical_and(msk, pos < CAP)
            plsc.store_scatter(candk, [pos], k, mask=msk)
            plsc.store_scatter(candi, [pos], i * L + lanes, mask=msk)
            return ptr_v + pcv
        ptr_v = lax.fori_loop(0, NCH, cstep, jnp.zeros((L,), jnp.int32),
                              unroll=4)
        ptr = jnp.minimum(jnp.min(ptr_v), jnp.int32(CAP))
        plsc.store_scatter(candk, [ptr + lanes], jnp.full((L,), BIGKEY))
        nv = jnp.right_shift(ptr + L - 1, 4)

        def e0(v, c):
            vm0, vm1 = c
            mv = jnp.min(candk[pl.ds(v * L, L)])
            vm0 = jnp.where(lanes == v, mv, vm0)
            vm1 = jnp.where(lanes == v - L, mv, vm1)
            return vm0, vm1
        vm = lax.fori_loop(0, nv, e0,
                           (jnp.full((L,), BIGKEY), jnp.full((L,), BIGKEY)))

        def estep(k, c):
            vm0, vm1 = c
            m = jnp.min(jnp.minimum(vm0, vm1))
            vstar = jnp.min(jnp.minimum(
                jnp.where(vm0 == m, lanes, jnp.int32(2 * L)),
                jnp.where(vm1 == m, lanes + L, jnp.int32(2 * L))))
            kv = candk[pl.ds(vstar * L, L)]
            iv = candi[pl.ds(vstar * L, L)]
            eq = kv == m
            im = jnp.min(jnp.where(eq, iv, jnp.int32(BIGKEY)))
            kv2 = jnp.where(jnp.logical_and(eq, iv == im), jnp.int32(BIGKEY), kv)
            candk[pl.ds(vstar * L, L)] = kv2
            nm = jnp.min(kv2)
            vm0 = jnp.where(lanes == vstar, nm, vm0)
            vm1 = jnp.where(lanes == vstar - L, nm, vm1)
            plsc.store_scatter(seli, [jnp.full((L,), k)],
                               jnp.full((L,), im), mask=lanes == 0)
            return vm0, vm1
        lax.fori_loop(0, KNN, estep, vm)

        cqx = plsc.load_gather(cq0, [jnp.full((L,), jj)])
        cqy = plsc.load_gather(cq1, [jnp.full((L,), jj)])
        cqz = plsc.load_gather(cq2, [jnp.full((L,), jj)])
        for g in range(KNN // L):
            idxv = seli[pl.ds(g * L, L)]
            pos = (g * L + lanes) * 3
            plsc.store_scatter(outrow, [pos],
                               plsc.load_gather(xb0, [idxv]) - cqx)
            plsc.store_scatter(outrow, [pos + 1],
                               plsc.load_gather(xb1, [idxv]) - cqy)
            plsc.store_scatter(outrow, [pos + 2],
                               plsc.load_gather(xb2, [idxv]) - cqz)
        pltpu.make_async_copy(outrow, p_hbm.at[r0 + jj], semx).start()
        pltpu.make_async_copy(outrow, p_hbm.at[r0 + jj], semx).wait()

    def rowpair(j2, _):
        jj0 = j2 * 2
        pltpu.make_async_copy(keys_hbm.at[r0 + jj0], keyb0, sem0).wait()

        @pl.when(jj0 + 1 < RPW)
        def _():
            pltpu.make_async_copy(keys_hbm.at[r0 + jj0 + 1], keyb1, sem1).start()
        process_row(jj0, keyb0)

        pltpu.make_async_copy(keys_hbm.at[r0 + jj0 + 1], keyb1, sem1).wait()

        @pl.when(jj0 + 2 < RPW)
        def _():
            pltpu.make_async_copy(keys_hbm.at[r0 + jj0 + 2], keyb0, sem0).start()
        process_row(jj0 + 1, keyb1)
        return 0

    lax.fori_loop(0, RPW // 2, rowpair, 0)


@jax.jit
def kernel(x):
    xt = jnp.transpose(x, (2, 0, 1))

    c_kb3 = pl.pallas_call(
        _fps_body,
        out_shape=jax.ShapeDtypeStruct((KC, B, 3), jnp.float32),
    )(xt)
    c = jnp.transpose(c_kb3, (1, 0, 2))

    nqb = KC // QB
    keys = pl.pallas_call(
        _keys_body,
        grid=(B, nqb),
        in_specs=[
            pl.BlockSpec((1, 3, N), lambda b, qb: (b, 0, 0)),
            pl.BlockSpec((1, QB, 3), lambda b, qb: (b, qb, 0)),
        ],
        out_specs=pl.BlockSpec((QB, N), lambda b, qb: (b * nqb + qb, 0)),
        out_shape=jax.ShapeDtypeStruct((NROWS, N), jnp.int32),
    )(jnp.transpose(x, (0, 2, 1)), c)

    c3bk = jnp.transpose(c_kb3, (2, 1, 0))

    mesh = plsc.VectorSubcoreMesh(core_axis_name="c", subcore_axis_name="s")
    sc = functools.partial(
        pl.kernel, mesh=mesh,
        out_type=jax.ShapeDtypeStruct((NROWS, KNN * 3), jnp.float32),
        scratch_types=[
            pltpu.VMEM((N,), jnp.float32),
            pltpu.VMEM((N,), jnp.float32),
            pltpu.VMEM((N,), jnp.float32),
            pltpu.VMEM((RPW,), jnp.float32),
            pltpu.VMEM((RPW,), jnp.float32),
            pltpu.VMEM((RPW,), jnp.float32),
            pltpu.VMEM((N,), jnp.int32),
            pltpu.VMEM((N,), jnp.int32),
            pltpu.VMEM((NBKT,), jnp.int32),
            pltpu.VMEM((CAP + L,), jnp.int32),
            pltpu.VMEM((CAP + L,), jnp.int32),
            pltpu.VMEM((KNN,), jnp.int32),
            pltpu.VMEM((KNN * 3,), jnp.float32),
            pltpu.SemaphoreType.DMA,
            pltpu.SemaphoreType.DMA,
            pltpu.SemaphoreType.DMA,
        ],
        compiler_params=pltpu.CompilerParams(needs_layout_passes=False),
    )(_sc_select)
    prows = sc(keys, xt, c3bk)

    p = prows.reshape(B, KC, KNN, 3)
    return (p, c)


# speedup vs baseline: 1.4808x; 1.0375x over previous
import functools
import jax
import jax.numpy as jnp
from jax import lax
from jax.experimental import pallas as pl
from jax.experimental.pallas import tpu as pltpu
from jax.experimental.pallas import tpu_sc as plsc

B, N, D = 16, 8192, 3
KC = 128
KNN = 64
QB = 8
NROWS = B * KC
NSUB = 32
RPW = NROWS // NSUB
NBKT = 2048
CAP = 512
L = 16
NCH = N // L
BIGKEY = 0x7FFFFFFF


def _fps_body(xt_ref, c_ref):
    X = xt_ref[0]
    Y = xt_ref[1]
    Z = xt_ref[2]
    iota = jax.lax.broadcasted_iota(jnp.int32, (B, N), 1)

    px = X[:, 0:1]
    py = Y[:, 0:1]
    pz = Z[:, 0:1]
    c_ref[0:1] = jnp.concatenate([px, py, pz], axis=1).reshape(1, B, 3)

    dists0 = jnp.full((B, N), jnp.inf, dtype=jnp.float32)

    def step(k, carry):
        dists, px, py, pz = carry
        d = (X - px) ** 2 + (Y - py) ** 2 + (Z - pz) ** 2
        dists = jnp.minimum(dists, d)
        m = jnp.max(dists, axis=1, keepdims=True)
        idx = jnp.min(jnp.where(dists == m, iota, N), axis=1, keepdims=True)
        onehot = iota == idx
        px = jnp.sum(jnp.where(onehot, X, 0.0), axis=1, keepdims=True)
        py = jnp.sum(jnp.where(onehot, Y, 0.0), axis=1, keepdims=True)
        pz = jnp.sum(jnp.where(onehot, Z, 0.0), axis=1, keepdims=True)
        c_ref[pl.ds(k, 1)] = jnp.concatenate([px, py, pz], axis=1).reshape(1, B, 3)
        return dists, px, py, pz

    jax.lax.fori_loop(1, KC, step, (dists0, px, py, pz))


def _keys_body(xt_ref, c_ref, k_ref):
    X = xt_ref[0, 0:1, :]
    Y = xt_ref[0, 1:2, :]
    Z = xt_ref[0, 2:3, :]
    cx = c_ref[0, :, 0:1]
    cy = c_ref[0, :, 1:2]
    cz = c_ref[0, :, 2:3]

    def rb(v):
        return v.astype(jnp.bfloat16).astype(jnp.float32)
    cn = cx * cx + cy * cy + cz * cz
    xn = X * X + Y * Y + Z * Z
    dot = rb(cx) * rb(X) + rb(cy) * rb(Y) + rb(cz) * rb(Z)
    d2 = cn + xn - 2.0 * dot

    s = jax.lax.bitcast_convert_type(d2, jnp.int32)
    m = jnp.right_shift(s, 31)
    k_ref[...] = s ^ (m & BIGKEY)


def _sc_select(keys_hbm, xt_hbm, c_hbm, p_hbm,
               xb0, xb1, xb2, cq0, cq1, cq2,
               keyb0, keyb1, hist, candk, candi, seli, outrow,
               sem0, sem1, semx):
    cid = lax.axis_index("c")
    sid = lax.axis_index("s")
    w = sid * 2 + cid
    b = w // 2
    q0 = (w % 2) * RPW
    r0 = b * KC + q0

    lanes = jax.lax.broadcasted_iota(jnp.int32, (L,), 0)
    ones16 = jnp.ones((L,), jnp.int32)

    pltpu.make_async_copy(xt_hbm.at[0, b], xb0, semx).start()
    pltpu.make_async_copy(xt_hbm.at[0, b], xb0, semx).wait()
    pltpu.make_async_copy(xt_hbm.at[1, b], xb1, semx).start()
    pltpu.make_async_copy(xt_hbm.at[1, b], xb1, semx).wait()
    pltpu.make_async_copy(xt_hbm.at[2, b], xb2, semx).start()
    pltpu.make_async_copy(xt_hbm.at[2, b], xb2, semx).wait()
    pltpu.make_async_copy(c_hbm.at[0, b, pl.ds(q0, RPW)], cq0, semx).start()
    pltpu.make_async_copy(c_hbm.at[0, b, pl.ds(q0, RPW)], cq0, semx).wait()
    pltpu.make_async_copy(c_hbm.at[1, b, pl.ds(q0, RPW)], cq1, semx).start()
    pltpu.make_async_copy(c_hbm.at[1, b, pl.ds(q0, RPW)], cq1, semx).wait()
    pltpu.make_async_copy(c_hbm.at[2, b, pl.ds(q0, RPW)], cq2, semx).start()
    pltpu.make_async_copy(c_hbm.at[2, b, pl.ds(q0, RPW)], cq2, semx).wait()

    pltpu.make_async_copy(keys_hbm.at[r0], keyb0, sem0).start()

    def process_row(jj, keyrow):
        def zstep(i, _):
            hist[pl.ds(i * L, L)] = jnp.zeros((L,), jnp.int32)
            return 0
        lax.fori_loop(0, NBKT // L, zstep, 0, unroll=8)

        def astep(i, mk):
            k = keyrow[pl.ds(i * L, L)]
            bkt = jnp.right_shift(k, 21) + 1024
            plsc.addupdate_scatter(hist, [bkt], ones16)
            return jnp.minimum(mk, k)
        mk = lax.fori_loop(0, NCH, astep, jnp.full((L,), BIGKEY), unroll=8)
        minkey = jnp.min(mk)
        gstart = jnp.right_shift(jnp.right_shift(minkey, 21) + 1024, 4)

        def wcond(c):
            return jnp.logical_not(c[2])

        def wbody(c):
            g, cum, _ = c
            sv = jnp.sum(hist[pl.ds(g * L, L)])
            hit = jnp.logical_or(cum + sv >= KNN, g >= NBKT // L - 1)
            return (jnp.where(hit, g, g + 1), jnp.where(hit, cum, cum + sv), hit)
        gstar, cumb, _ = lax.while_loop(
            wcond, wbody, (gstart, jnp.int32(0), False))
        vg = hist[pl.ds(gstar * L, L)]
        cin = plsc.cumsum(vg) + cumb
        crossed = cin >= KNN
        lane_b = jnp.min(jnp.where(crossed, lanes, jnp.int32(L)))
        bstar = gstar * L + jnp.minimum(lane_b, L - 1)

        def cstep(i, ptr_v):
            k = keyrow[pl.ds(i * L, L)]
            bkt = jnp.right_shift(k, 21) + 1024
            msk = bkt <= bstar
            pcv = plsc.all_reduce_population_count(msk)
            pos = ptr_v + plsc.cumsum(msk.astype(jnp.int32)) - 1
            msk = jnp.logical_and(msk, pos < CAP)
            plsc.store_scatter(candk, [pos], k, mask=msk)
            plsc.store_scatter(candi, [pos], i * L + lanes, mask=msk)
            return ptr_v + pcv
        ptr_v = lax.fori_loop(0, NCH, cstep, jnp.zeros((L,), jnp.int32),
                              unroll=4)
        ptr = jnp.minimum(jnp.min(ptr_v), jnp.int32(CAP))
        plsc.store_scatter(candk, [ptr + lanes], jnp.full((L,), BIGKEY))
        nv = jnp.right_shift(ptr + L - 1, 4)

        def e0(v, c):
            vm0, vm1 = c
            mv = jnp.min(candk[pl.ds(v * L, L)])
            vm0 = jnp.where(lanes == v, mv, vm0)
            vm1 = jnp.where(lanes == v - L, mv, vm1)
            return vm0, vm1
        vm = lax.fori_loop(0, nv, e0,
                           (jnp.full((L,), BIGKEY), jnp.full((L,), BIGKEY)))

        def estep(k, c):
            vm0, vm1 = c
            m = jnp.min(jnp.minimum(vm0, vm1))
            eq0 = vm0 == m
            f0 = plsc.all_reduce_ffs(eq0)
            f1 = plsc.all_reduce_ffs(vm1 == m)
            any0 = plsc.all_reduce_population_count(eq0)
            vstar = jnp.where(any0 > 0, f0, f1 + L)
            vstar_s = jnp.min(vstar)
            kv = candk[pl.ds(vstar_s * L, L)]
            iv = candi[pl.ds(vstar_s * L, L)]
            klane = plsc.all_reduce_ffs(kv == m)
            kill = lanes == klane
            kv2 = jnp.where(kill, jnp.int32(BIGKEY), kv)
            candk[pl.ds(vstar_s * L, L)] = kv2
            plsc.store_scatter(seli, [jnp.full((L,), k)], iv, mask=kill)
            nm = jnp.min(kv2)
            vm0 = jnp.where(lanes == vstar, nm, vm0)
            vm1 = jnp.where(lanes == vstar - L, nm, vm1)
            return vm0, vm1
        lax.fori_loop(0, KNN, estep, vm)

        cqx = plsc.load_gather(cq0, [jnp.full((L,), jj)])
        cqy = plsc.load_gather(cq1, [jnp.full((L,), jj)])
        cqz = plsc.load_gather(cq2, [jnp.full((L,), jj)])
        for g in range(KNN // L):
            idxv = seli[pl.ds(g * L, L)]
            pos = (g * L + lanes) * 3
            plsc.store_scatter(outrow, [pos],
                               plsc.load_gather(xb0, [idxv]) - cqx)
            plsc.store_scatter(outrow, [pos + 1],
                               plsc.load_gather(xb1, [idxv]) - cqy)
            plsc.store_scatter(outrow, [pos + 2],
                               plsc.load_gather(xb2, [idxv]) - cqz)
        pltpu.make_async_copy(outrow, p_hbm.at[r0 + jj], semx).start()
        pltpu.make_async_copy(outrow, p_hbm.at[r0 + jj], semx).wait()

    def rowpair(j2, _):
        jj0 = j2 * 2
        pltpu.make_async_copy(keys_hbm.at[r0 + jj0], keyb0, sem0).wait()

        @pl.when(jj0 + 1 < RPW)
        def _():
            pltpu.make_async_copy(keys_hbm.at[r0 + jj0 + 1], keyb1, sem1).start()
        process_row(jj0, keyb0)

        pltpu.make_async_copy(keys_hbm.at[r0 + jj0 + 1], keyb1, sem1).wait()

        @pl.when(jj0 + 2 < RPW)
        def _():
            pltpu.make_async_copy(keys_hbm.at[r0 + jj0 + 2], keyb0, sem0).start()
        process_row(jj0 + 1, keyb1)
        return 0

    lax.fori_loop(0, RPW // 2, rowpair, 0)


@jax.jit
def kernel(x):
    xt = jnp.transpose(x, (2, 0, 1))

    c_kb3 = pl.pallas_call(
        _fps_body,
        out_shape=jax.ShapeDtypeStruct((KC, B, 3), jnp.float32),
    )(xt)
    c = jnp.transpose(c_kb3, (1, 0, 2))

    nqb = KC // QB
    keys = pl.pallas_call(
        _keys_body,
        grid=(B, nqb),
        in_specs=[
            pl.BlockSpec((1, 3, N), lambda b, qb: (b, 0, 0)),
            pl.BlockSpec((1, QB, 3), lambda b, qb: (b, qb, 0)),
        ],
        out_specs=pl.BlockSpec((QB, N), lambda b, qb: (b * nqb + qb, 0)),
        out_shape=jax.ShapeDtypeStruct((NROWS, N), jnp.int32),
    )(jnp.transpose(x, (0, 2, 1)), c)

    c3bk = jnp.transpose(c_kb3, (2, 1, 0))

    mesh = plsc.VectorSubcoreMesh(core_axis_name="c", subcore_axis_name="s")
    sc = functools.partial(
        pl.kernel, mesh=mesh,
        out_type=jax.ShapeDtypeStruct((NROWS, KNN * 3), jnp.float32),
        scratch_types=[
            pltpu.VMEM((N,), jnp.float32),
            pltpu.VMEM((N,), jnp.float32),
            pltpu.VMEM((N,), jnp.float32),
            pltpu.VMEM((RPW,), jnp.float32),
            pltpu.VMEM((RPW,), jnp.float32),
            pltpu.VMEM((RPW,), jnp.float32),
            pltpu.VMEM((N,), jnp.int32),
            pltpu.VMEM((N,), jnp.int32),
            pltpu.VMEM((NBKT,), jnp.int32),
            pltpu.VMEM((CAP + L,), jnp.int32),
            pltpu.VMEM((CAP + L,), jnp.int32),
            pltpu.VMEM((KNN,), jnp.int32),
            pltpu.VMEM((KNN * 3,), jnp.float32),
            pltpu.SemaphoreType.DMA,
            pltpu.SemaphoreType.DMA,
            pltpu.SemaphoreType.DMA,
        ],
        compiler_params=pltpu.CompilerParams(needs_layout_passes=False),
    )(_sc_select)
    prows = sc(keys, xt, c3bk)

    p = prows.reshape(B, KC, KNN, 3)
    return (p, c)
